# baseline (speedup 1.0000x reference)
import functools

import jax
import jax.numpy as jnp
from jax import lax
from jax.experimental import pallas as pl
from jax.experimental.pallas import tpu as pltpu
from jax.experimental.pallas import tpu_sc as plsc

NPAD = 1024
IDXW = 208
NW = 32
RPW = NPAD // NW
OUTR = 56


def _pyr_body(f_ref, out_ref):
    neg = jnp.float32(-jnp.inf)
    f = f_ref[0]
    w1 = jnp.maximum(f[0:31], f[1:32])
    w2 = jnp.maximum(w1[0:29], w1[2:31])
    for kw, x in ((0, f), (1, w1), (2, w2)):
        h1 = jnp.maximum(x[:, 0:31], x[:, 1:32])
        h2 = jnp.maximum(h1[:, 0:29], h1[:, 2:31])
        for kh, y in ((0, x), (1, h1), (2, h2)):
            lvl = kh * 3 + kw
            out_ref[lvl, 0, 0:y.shape[0], 0:y.shape[1]] = y
            out_ref[lvl, 0, 32] = jnp.full((33, y.shape[2]), neg)
            out_ref[lvl, 0, :, 32] = jnp.full((33, y.shape[2]), neg)


def _build_pyramid(fT):
    B, W, H, C = fT.shape
    return pl.pallas_call(
        _pyr_body,
        grid=(B,),
        in_specs=[pl.BlockSpec((1, W, H, C), lambda b: (b, 0, 0, 0))],
        out_specs=pl.BlockSpec((9, 1, 33, 33, C), lambda b: (0, b, 0, 0, 0)),
        out_shape=jax.ShapeDtypeStruct((9, B, 33, 33, C), jnp.float32),
    )(fT)


def _sc_pool_body(table_ref, idx_ref, out_ref,
                  idx_a, idx_b, rows_a, rows_b, out_v, sem_a, sem_b):
    wid = lax.axis_index("s") * 2 + lax.axis_index("c")
    neg = jnp.full((16,), -jnp.inf, jnp.float32)
    zero = jnp.zeros((16,), jnp.float32)

    def emit_bin(rows, r0, bi):
        for c in range(6):
            sl = pl.ds(c * 16, 16)
            v = jnp.maximum(
                jnp.maximum(rows[r0, sl], rows[r0 + 1, sl]),
                jnp.maximum(rows[r0 + 2, sl], rows[r0 + 3, sl]),
            )
            out_v[bi, sl] = jnp.where(v > neg, v, zero)

    def roi_body(r, carry):
        roi = wid * RPW + r
        pltpu.sync_copy(idx_ref.at[pl.ds(roi * IDXW, 104)], idx_a)
        pltpu.sync_copy(idx_ref.at[pl.ds(roi * IDXW + 104, 104)], idx_b)
        pltpu.async_copy(table_ref.at[idx_a], rows_a, sem_a).wait()
        pltpu.async_copy(table_ref.at[idx_b], rows_b, sem_b).wait()

        def bin_a(bi, c2):
            emit_bin(rows_a, bi * 4, bi)
            return c2

        def bin_b(j, c2):
            emit_bin(rows_b, j * 4, 26 + j)
            return c2

        lax.fori_loop(0, 26, bin_a, 0)
        lax.fori_loop(0, 23, bin_b, 0)
        pltpu.sync_copy(out_v, out_ref.at[pl.ds(roi * OUTR, OUTR)])
        return carry

    lax.fori_loop(0, RPW, roi_body, 0)


def _bin_lookup_params(lo, hi):
    ln = hi - lo
    k = jnp.where(ln >= 4, 2, jnp.where(ln >= 2, 1, 0))
    p = jnp.int32(1) << k
    o1 = jnp.clip(lo, 0, 32 - p)
    o2 = jnp.clip(hi - p, 0, 32 - p)
    o1 = jnp.where(ln > 0, o1, 32)
    o2 = jnp.where(ln > 0, o2, 32)
    return k, o1, o2


def _make_indices(rois):
    N = rois.shape[0]
    rois_i = rois.astype(jnp.int32)
    start = rois_i[:, 1:3].astype(jnp.float32)
    end = rois_i[:, 3:5].astype(jnp.float32)
    size = jnp.maximum(end - start, 1.0) / 7.0
    i = jnp.arange(7, dtype=jnp.float32)
    lo = jnp.clip(jnp.floor(i[None, :, None] * size[:, None, :] + start[:, None, :]), 0, 32).astype(jnp.int32)
    hi = jnp.clip(jnp.ceil((i[None, :, None] + 1.0) * size[:, None, :] + start[:, None, :]), 0, 32).astype(jnp.int32)
    k, o1, o2 = _bin_lookup_params(lo, hi)
    b = rois_i[:, 0]
    kw, o1w, o2w = k[:, :, 0], o1[:, :, 0], o2[:, :, 0]
    kh, o1h, o2h = k[:, :, 1], o1[:, :, 1], o2[:, :, 1]
    lvl = kh[:, :, None] * 3 + kw[:, None, :]
    base = (lvl * 32 + b[:, None, None]) * 33
    ws = jnp.stack([o1w, o1w, o2w, o2w], -1)[:, None, :, :]
    hs = jnp.stack([o1h, o2h, o1h, o2h], -1)[:, :, None, :]
    idx = ((base[..., None] + ws) * 33 + hs).reshape(N, 196)
    idx = jnp.concatenate([idx, jnp.zeros((N, IDXW - 196), jnp.int32)], axis=1)
    if N < NPAD:
        idx = jnp.concatenate([idx, jnp.zeros((NPAD - N, IDXW), jnp.int32)], 0)
    return idx.reshape(-1)


def kernel(features, rois):
    N = rois.shape[0]
    C = features.shape[1]
    fT = jnp.transpose(features, (0, 3, 2, 1))
    fP = jnp.pad(fT, ((0, 0), (0, 0), (0, 0), (0, 128 - C)))
    pyr = _build_pyramid(fP)
    table = pyr.reshape(9 * 32 * 33 * 33, 128)
    idx = _make_indices(rois)

    sc_pool = functools.partial(
        pl.kernel,
        out_type=jax.ShapeDtypeStruct((NPAD * OUTR, C), jnp.float32),
        mesh=plsc.VectorSubcoreMesh(core_axis_name="c", subcore_axis_name="s"),
        scratch_types=[
            pltpu.VMEM((104,), jnp.int32),
            pltpu.VMEM((104,), jnp.int32),
            pltpu.VMEM((104, 128), jnp.float32),
            pltpu.VMEM((104, 128), jnp.float32),
            pltpu.VMEM((OUTR, C), jnp.float32),
            pltpu.SemaphoreType.DMA,
            pltpu.SemaphoreType.DMA,
        ],
    )(_sc_pool_body)
    out = sc_pool(table, idx)
    pooled = out.reshape(NPAD, OUTR, C)[:N, :49]
    return jnp.transpose(pooled, (0, 2, 1)).reshape(N, C, 7, 7)


# speedup vs baseline: 1.0468x; 1.0468x over previous
import functools

import jax
import jax.numpy as jnp
from jax import lax
from jax.experimental import pallas as pl
from jax.experimental.pallas import tpu as pltpu
from jax.experimental.pallas import tpu_sc as plsc

NPAD = 1024
IDXW = 208
NW = 32
RPW = NPAD // NW
OUTR = 56


def _pyr_body(f_ref, out_ref):
    neg = jnp.float32(-jnp.inf)
    f = f_ref[0]
    w1 = jnp.maximum(f[0:31], f[1:32])
    w2 = jnp.maximum(w1[0:29], w1[2:31])
    for kw, x in ((0, f), (1, w1), (2, w2)):
        h1 = jnp.maximum(x[:, 0:31], x[:, 1:32])
        h2 = jnp.maximum(h1[:, 0:29], h1[:, 2:31])
        for kh, y in ((0, x), (1, h1), (2, h2)):
            lvl = kh * 3 + kw
            out_ref[lvl, 0, 0:y.shape[0], 0:y.shape[1]] = y
            out_ref[lvl, 0, 32] = jnp.full((33, y.shape[2]), neg)
            out_ref[lvl, 0, :, 32] = jnp.full((33, y.shape[2]), neg)


def _build_pyramid(fT):
    B, W, H, C = fT.shape
    return pl.pallas_call(
        _pyr_body,
        grid=(B,),
        in_specs=[pl.BlockSpec((1, W, H, C), lambda b: (b, 0, 0, 0))],
        out_specs=pl.BlockSpec((9, 1, 33, 33, C), lambda b: (0, b, 0, 0, 0)),
        out_shape=jax.ShapeDtypeStruct((9, B, 33, 33, C), jnp.float32),
    )(fT)


def _sc_pool_body(table_ref, idx_ref, out_ref,
                  idx_all, rows_a1, rows_a2, rows_b1, rows_b2, out_v,
                  sem_a1, sem_a2, sem_b1, sem_b2):
    wid = lax.axis_index("s") * 2 + lax.axis_index("c")
    neg = jnp.full((16,), -jnp.inf, jnp.float32)
    zero = jnp.zeros((16,), jnp.float32)
    pltpu.sync_copy(idx_ref.at[pl.ds(wid * (RPW * IDXW), RPW * IDXW)], idx_all)

    def fire(r, r1, r2, s1, s2):
        c1 = pltpu.async_copy(table_ref.at[idx_all.at[pl.ds(r * IDXW, 104)]], r1, s1)
        c2 = pltpu.async_copy(
            table_ref.at[idx_all.at[pl.ds(r * IDXW + 104, 104)]], r2, s2)
        return c1, c2

    def emit(r1, r2, slot):
        def bin1(bi, c2):
            base = slot * OUTR + bi
            for c in range(6):
                sl = pl.ds(c * 16, 16)
                v = jnp.maximum(
                    jnp.maximum(r1[bi * 4, sl], r1[bi * 4 + 1, sl]),
                    jnp.maximum(r1[bi * 4 + 2, sl], r1[bi * 4 + 3, sl]),
                )
                out_v[base, sl] = jnp.where(v > neg, v, zero)
            return c2

        def bin2(j, c2):
            base = slot * OUTR + 26 + j
            for c in range(6):
                sl = pl.ds(c * 16, 16)
                v = jnp.maximum(
                    jnp.maximum(r2[j * 4, sl], r2[j * 4 + 1, sl]),
                    jnp.maximum(r2[j * 4 + 2, sl], r2[j * 4 + 3, sl]),
                )
                out_v[base, sl] = jnp.where(v > neg, v, zero)
            return c2

        lax.fori_loop(0, 26, bin1, 0)
        lax.fori_loop(0, 23, bin2, 0)

    fire(0, rows_a1, rows_a2, sem_a1, sem_a2)

    def step(i, carry):
        ra = 2 * i
        rb = 2 * i + 1
        pltpu.make_async_copy(table_ref, rows_a1, sem_a1).wait()
        pltpu.make_async_copy(table_ref, rows_a2, sem_a2).wait()
        fire(rb, rows_b1, rows_b2, sem_b1, sem_b2)
        emit(rows_a1, rows_a2, ra % 8)
        pltpu.make_async_copy(table_ref, rows_b1, sem_b1).wait()
        pltpu.make_async_copy(table_ref, rows_b2, sem_b2).wait()
        fire(jnp.minimum(ra + 2, RPW - 2), rows_a1, rows_a2, sem_a1, sem_a2)
        emit(rows_b1, rows_b2, rb % 8)

        @pl.when(i % 4 == 3)
        def _flush():
            base = (wid * RPW + (i - 3) * 2) * OUTR
            pltpu.sync_copy(out_v, out_ref.at[pl.ds(base, 8 * OUTR)])

        return carry

    lax.fori_loop(0, RPW // 2, step, 0)
    pltpu.make_async_copy(table_ref, rows_a1, sem_a1).wait()
    pltpu.make_async_copy(table_ref, rows_a2, sem_a2).wait()


def _bin_lookup_params(lo, hi):
    ln = hi - lo
    k = jnp.where(ln >= 4, 2, jnp.where(ln >= 2, 1, 0))
    p = jnp.int32(1) << k
    o1 = jnp.clip(lo, 0, 32 - p)
    o2 = jnp.clip(hi - p, 0, 32 - p)
    o1 = jnp.where(ln > 0, o1, 32)
    o2 = jnp.where(ln > 0, o2, 32)
    return k, o1, o2


def _make_indices(rois):
    N = rois.shape[0]
    rois_i = rois.astype(jnp.int32)
    start = rois_i[:, 1:3].astype(jnp.float32)
    end = rois_i[:, 3:5].astype(jnp.float32)
    size = jnp.maximum(end - start, 1.0) / 7.0
    i = jnp.arange(7, dtype=jnp.float32)
    lo = jnp.clip(jnp.floor(i[None, :, None] * size[:, None, :] + start[:, None, :]), 0, 32).astype(jnp.int32)
    hi = jnp.clip(jnp.ceil((i[None, :, None] + 1.0) * size[:, None, :] + start[:, None, :]), 0, 32).astype(jnp.int32)
    k, o1, o2 = _bin_lookup_params(lo, hi)
    b = rois_i[:, 0]
    kw, o1w, o2w = k[:, :, 0], o1[:, :, 0], o2[:, :, 0]
    kh, o1h, o2h = k[:, :, 1], o1[:, :, 1], o2[:, :, 1]
    lvl = kh[:, :, None] * 3 + kw[:, None, :]
    base = (lvl * 32 + b[:, None, None]) * 33
    ws = jnp.stack([o1w, o1w, o2w, o2w], -1)[:, None, :, :]
    hs = jnp.stack([o1h, o2h, o1h, o2h], -1)[:, :, None, :]
    idx = ((base[..., None] + ws) * 33 + hs).reshape(N, 196)
    idx = jnp.concatenate([idx, jnp.zeros((N, IDXW - 196), jnp.int32)], axis=1)
    if N < NPAD:
        idx = jnp.concatenate([idx, jnp.zeros((NPAD - N, IDXW), jnp.int32)], 0)
    return idx.reshape(-1)


def kernel(features, rois):
    N = rois.shape[0]
    C = features.shape[1]
    fT = jnp.transpose(features, (0, 3, 2, 1))
    fP = jnp.pad(fT, ((0, 0), (0, 0), (0, 0), (0, 128 - C)))
    pyr = _build_pyramid(fP)
    table = pyr.reshape(9 * 32 * 33 * 33, 128)
    idx = _make_indices(rois)

    sc_pool = functools.partial(
        pl.kernel,
        out_type=jax.ShapeDtypeStruct((NPAD * OUTR, C), jnp.float32),
        mesh=plsc.VectorSubcoreMesh(core_axis_name="c", subcore_axis_name="s"),
        scratch_types=[
            pltpu.VMEM((RPW * IDXW,), jnp.int32),
            pltpu.VMEM((104, 128), jnp.float32),
            pltpu.VMEM((104, 128), jnp.float32),
            pltpu.VMEM((104, 128), jnp.float32),
            pltpu.VMEM((104, 128), jnp.float32),
            pltpu.VMEM((8 * OUTR, C), jnp.float32),
            pltpu.SemaphoreType.DMA,
            pltpu.SemaphoreType.DMA,
            pltpu.SemaphoreType.DMA,
            pltpu.SemaphoreType.DMA,
        ],
    )(_sc_pool_body)
    out = sc_pool(table, idx)
    pooled = out.reshape(NPAD, OUTR, C)[:N, :49]
    return jnp.transpose(pooled, (0, 2, 1)).reshape(N, C, 7, 7)


# speedup vs baseline: 5.8073x; 5.5476x over previous
import jax
import jax.numpy as jnp
from jax.experimental import pallas as pl
from jax.experimental.pallas import tpu as pltpu

OUT_H = 7
OUT_W = 7
WIN = 6
K = 40


def _roi_pool_body(params_ref, f_ref, t_ref, out_ref, p_ref, tmp_ref, sem):
    neg = jnp.float32(-jnp.inf)

    @pl.when(pl.program_id(0) == 0)
    def _build_pyramid():
        cp = pltpu.make_async_copy(f_ref, p_ref.at[0, :, 0:32], sem)
        cp.start()
        cp.wait()
        p_ref[1, :, 0:31] = jnp.maximum(p_ref[0, :, 0:31], p_ref[0, :, 1:32])
        p_ref[1, :, 31] = p_ref[0, :, 31]
        p_ref[2, :, 0:29] = jnp.maximum(p_ref[1, :, 0:29], p_ref[1, :, 2:31])
        p_ref[:, :, 32] = jnp.full((3, 32, 32, 96), neg)

    for k in range(K):
        b = params_ref[k, 0]
        for wb in range(OUT_W):
            kw = params_ref[k, 1 + wb]
            o1 = params_ref[k, 8 + wb]
            o2 = params_ref[k, 15 + wb]
            tmp_ref[k, wb] = jnp.maximum(p_ref[kw, b, o1], p_ref[kw, b, o2])
    for k in range(K):
        for hb in range(OUT_H):
            s0 = params_ref[k, 29 + hb]
            mi = params_ref[k, 36 + hb]
            win = tmp_ref[k, :, pl.ds(s0, WIN), :]
            row = jnp.max(win + t_ref[mi][None], axis=1)
            out_ref[k, hb] = jnp.where(row > neg, row, jnp.float32(0.0))


def _bin_params(rois):
    rois_i = rois.astype(jnp.int32)
    start_w = rois_i[:, 1].astype(jnp.float32)
    start_h = rois_i[:, 2].astype(jnp.float32)
    end_w = rois_i[:, 3].astype(jnp.float32)
    end_h = rois_i[:, 4].astype(jnp.float32)
    bin_h = jnp.maximum(end_h - start_h, 1.0) / float(OUT_H)
    bin_w = jnp.maximum(end_w - start_w, 1.0) / float(OUT_W)
    hs = jnp.arange(OUT_H, dtype=jnp.float32)
    ws = jnp.arange(OUT_W, dtype=jnp.float32)
    clip = lambda a: jnp.clip(a, 0, 32).astype(jnp.int32)
    h_start = clip(jnp.floor(hs[None, :] * bin_h[:, None] + start_h[:, None]))
    h_end = clip(jnp.ceil((hs[None, :] + 1.0) * bin_h[:, None] + start_h[:, None]))
    w_start = clip(jnp.floor(ws[None, :] * bin_w[:, None] + start_w[:, None]))
    w_end = clip(jnp.ceil((ws[None, :] + 1.0) * bin_w[:, None] + start_w[:, None]))
    wlen = w_end - w_start
    kw = jnp.where(wlen >= 4, 2, jnp.where(wlen >= 2, 1, 0))
    pw = jnp.int32(1) << kw
    o1 = jnp.clip(w_start, 0, 32 - pw)
    o2 = jnp.clip(w_end - pw, 0, 32 - pw)
    o1 = jnp.where(wlen > 0, o1, 32)
    o2 = jnp.where(wlen > 0, o2, 32)
    valid = (wlen > 0).astype(jnp.int32)
    hs0 = jnp.clip(h_start, 0, 32 - WIN)
    mi = (h_start - hs0) * 7 + (h_end - hs0)
    return jnp.concatenate(
        [
            rois_i[:, :1],
            kw, o1, o2, valid, hs0, mi,
            jnp.zeros((rois_i.shape[0], 21), jnp.int32),
        ],
        axis=1,
    )


def kernel(features, rois):
    N = rois.shape[0]
    C = features.shape[1]
    params = _bin_params(rois)
    fT = jnp.transpose(features, (0, 3, 2, 1))
    d = jnp.arange(WIN, dtype=jnp.int32)
    off = jnp.arange(49, dtype=jnp.int32) // 7
    end = jnp.arange(49, dtype=jnp.int32) % 7
    tbl = jnp.where(
        (d[None, :] >= off[:, None]) & (d[None, :] < end[:, None]),
        jnp.float32(0.0), jnp.float32(-jnp.inf),
    )
    tbl = jnp.broadcast_to(tbl[:, :, None], (49, WIN, C))
    out = pl.pallas_call(
        _roi_pool_body,
        grid=(N // K,),
        in_specs=[
            pl.BlockSpec((K, 64), lambda i: (i, 0), memory_space=pltpu.SMEM),
            pl.BlockSpec(memory_space=pltpu.MemorySpace.HBM),
            pl.BlockSpec((49, WIN, C), lambda i: (0, 0, 0)),
        ],
        out_specs=pl.BlockSpec((K, OUT_H, OUT_W, C), lambda i: (i, 0, 0, 0)),
        out_shape=jax.ShapeDtypeStruct((N, OUT_H, OUT_W, C), jnp.float32),
        scratch_shapes=[
            pltpu.VMEM((3, 32, 33, 32, C), jnp.float32),
            pltpu.VMEM((K, OUT_W, 32, C), jnp.float32),
            pltpu.SemaphoreType.DMA,
        ],
        compiler_params=pltpu.CompilerParams(
            dimension_semantics=("arbitrary",),
        ),
    )(params, fT, tbl)
    return jnp.transpose(out, (0, 3, 1, 2))
